# baseline (speedup 1.0000x reference)
import functools

import jax
import jax.numpy as jnp
from jax import lax
from jax.experimental import pallas as pl
from jax.experimental.pallas import tpu as pltpu
from jax.experimental.pallas import tpu_sc as plsc

_D = 768
_NC = 2
_NS = 16
_NW = _NC * _NS
_LG = 2


def _gather_rows(idx1, table, batch, seq):
    bat_per_w = batch // _NW
    nsteps = seq // _LG
    rows_per_g = _LG * bat_per_w
    mesh = plsc.VectorSubcoreMesh(core_axis_name="c", subcore_axis_name="s")

    @functools.partial(
        pl.kernel,
        mesh=mesh,
        out_type=jax.ShapeDtypeStruct((seq, batch, _D), jnp.float32),
        scratch_types=[
            pltpu.VMEM((seq * bat_per_w,), jnp.int32),
            pltpu.VMEM((rows_per_g, _D), jnp.float32),
            pltpu.VMEM((rows_per_g, _D), jnp.float32),
            pltpu.SemaphoreType.DMA,
            pltpu.SemaphoreType.DMA,
        ],
    )
    def k(idx_hbm, table_hbm, out_hbm, idx_v, rows0_v, rows1_v, gsem, osem):
        wid = lax.axis_index("s") * _NC + lax.axis_index("c")
        base_b = wid * bat_per_w
        pltpu.sync_copy(
            idx_hbm.at[pl.ds(wid * seq * bat_per_w, seq * bat_per_w)], idx_v)

        bufs = (rows0_v, rows1_v)
        nb = len(bufs)
        gd = [None] * nsteps
        od = [None] * nsteps
        gd[0] = pltpu.async_copy(
            table_hbm.at[idx_v.at[pl.ds(0, rows_per_g)]], bufs[0], gsem)
        for t in range(nsteps):
            buf = bufs[t % nb]
            gd[t].wait()
            ws = []
            for j in range(_LG):
                ws.append(pltpu.async_copy(
                    buf.at[pl.ds(j * bat_per_w, bat_per_w)],
                    out_hbm.at[t * _LG + j].at[pl.ds(base_b, bat_per_w)],
                    osem))
            od[t] = ws
            if t + 1 < nsteps:
                if t >= 1:
                    for w in od[t - 1]:
                        w.wait()
                gd[t + 1] = pltpu.async_copy(
                    table_hbm.at[
                        idx_v.at[pl.ds((t + 1) * rows_per_g, rows_per_g)]],
                    bufs[(t + 1) % nb], gsem)
        for t in (nsteps - 2, nsteps - 1):
            for w in od[t]:
                w.wait()

    return k(idx1, table)


def kernel(input_ids, table):
    b, s = input_ids.shape
    bat_per_w = b // _NW
    idx1 = (input_ids.astype(jnp.int32).T
            .reshape(s, _NW, bat_per_w)
            .transpose(1, 0, 2)
            .reshape(-1))
    out_t = _gather_rows(idx1, table, b, s)
    return jnp.transpose(out_t, (1, 0, 2))
